# Initial kernel scaffold; baseline (speedup 1.0000x reference)
import jax
import jax.numpy as jnp
from jax.experimental import pallas as pl


def kernel(indices, token_table, first_order_table, first_order_bias):
    raise NotImplementedError("write your pallas kernel here")


# speedup vs baseline: 1.2012x; 1.2012x over previous
import functools

import jax
import jax.numpy as jnp
from jax import lax
from jax.experimental import pallas as pl
from jax.experimental.pallas import tpu as pltpu
from jax.experimental.pallas import tpu_sc as plsc

B, F, V, D = 16384, 26, 1000000, 16
OUT_W = F * D + 1
L = 16

NC, NS = 2, 16
NW = NC * NS
ROWS_PER_TILE = B // NW
CB = 64
NCHUNK = ROWS_PER_TILE // CB


def _tile_body(idx_hbm, tok_hbm, fo_hbm, bias_hbm, out_hbm,
               idx_v, rows_v, fo_v, out_v, bias_v, sem_tok, sem_fo):
    wid = lax.axis_index("s") * NC + lax.axis_index("c")
    tile_base = wid * ROWS_PER_TILE
    pltpu.sync_copy(bias_hbm, bias_v)
    bias_vec = bias_v[...]

    def chunk(c, carry):
        base = tile_base + c * CB
        pltpu.sync_copy(idx_hbm.at[pl.ds(base * F, CB * F)], idx_v)
        cp_tok = pltpu.async_copy(tok_hbm.at[idx_v], rows_v, sem_tok)
        cp_fo = pltpu.async_copy(fo_hbm.at[idx_v], fo_v, sem_fo)
        cp_tok.wait()
        cp_fo.wait()

        def row(b, carry2):
            for f in range(F):
                out_v[b, pl.ds(f * D, D)] = rows_v[b * F + f]
            return carry2

        lax.fori_loop(0, CB, row, 0, unroll=False)

        riota = lax.iota(jnp.int32, L)

        def grp(g, carry2):
            b0 = g * L
            acc = bias_vec
            for f in range(F):
                acc = acc + plsc.load_gather(fo_v, [(b0 + riota) * F + f])
            plsc.store_scatter(out_v, [b0 + riota, jnp.full((L,), F * D, jnp.int32)], acc)
            return carry2

        lax.fori_loop(0, CB // L, grp, 0, unroll=False)

        pltpu.sync_copy(out_v, out_hbm.at[pl.ds(base, CB)])
        return carry

    lax.fori_loop(0, NCHUNK, chunk, 0, unroll=False)


@jax.jit
def _run(idx_flat, token_table, first_order_table, first_order_bias):
    mesh = plsc.VectorSubcoreMesh(core_axis_name="c", subcore_axis_name="s",
                                  num_cores=NC, num_subcores=NS)
    k = functools.partial(
        pl.kernel,
        mesh=mesh,
        out_type=jax.ShapeDtypeStruct((B, OUT_W), jnp.float32),
        scratch_types=[
            pltpu.VMEM((CB * F,), jnp.int32),
            pltpu.VMEM((CB * F, D), jnp.float32),
            pltpu.VMEM((CB * F,), jnp.float32),
            pltpu.VMEM((CB, OUT_W), jnp.float32),
            pltpu.VMEM((L,), jnp.float32),
            pltpu.SemaphoreType.DMA,
            pltpu.SemaphoreType.DMA,
        ],
        compiler_params=pltpu.CompilerParams(
            needs_layout_passes=False, use_tc_tiling_on_sc=False),
    )(_tile_body)
    return k(idx_flat, token_table, first_order_table, first_order_bias)


def kernel(indices, token_table, first_order_table, first_order_bias):
    idx_flat = indices.reshape(-1)
    fo_flat = first_order_table.reshape(-1)
    bias16 = jnp.broadcast_to(first_order_bias, (L,))
    return _run(idx_flat, token_table, fo_flat, bias16)
